# Initial kernel scaffold; baseline (speedup 1.0000x reference)
import jax
import jax.numpy as jnp
from jax.experimental import pallas as pl


def kernel(param, scalar, noise, b):
    raise NotImplementedError("write your pallas kernel here")


# speedup vs baseline: 12.7785x; 12.7785x over previous
import jax
import jax.numpy as jnp
from jax.experimental import pallas as pl

_M = 64
_N = 4096
_D = 64


def _mask_kernel(c_ref, param_ref, noise_ref, out_ref):
    x = param_ref[0] + noise_ref[0] / jnp.float32(1000.0)
    xmax = jnp.max(x, axis=1, keepdims=True)
    e = jnp.exp(x - xmax)
    z = jnp.sum(e, axis=1, keepdims=True)
    p = e / z
    key = jax.lax.bitcast_convert_type(p, jnp.uint32)

    def body(_, carry):
        lo, hi = carry
        mid = lo + ((hi - lo) >> jnp.uint32(1))
        cnt = jnp.sum((key >= mid).astype(jnp.int32), axis=1, keepdims=True)
        pred = cnt >= _D
        return jnp.where(pred, mid, lo), jnp.where(pred, hi, mid)

    lo0 = jnp.zeros((_M, 1), jnp.uint32)
    hi0 = jnp.full((_M, 1), 0x40000000, jnp.uint32)
    thr, _ = jax.lax.fori_loop(0, 30, body, (lo0, hi0))

    gt = key > thr
    eq = key == thr
    r = _D - jnp.sum(gt.astype(jnp.int32), axis=1, keepdims=True)
    idx = jax.lax.broadcasted_iota(jnp.int32, (_M, _N), 1)
    eqi = eq.astype(jnp.int32)

    def body2(_, carry):
        lo, hi = carry
        mid = (lo + hi) >> 1
        c = jnp.sum(jnp.where(idx <= mid, eqi, 0), axis=1, keepdims=True)
        pred = c >= r
        return jnp.where(pred, lo, mid), jnp.where(pred, mid, hi)

    lo0i = jnp.full((_M, 1), -1, jnp.int32)
    hi0i = jnp.full((_M, 1), _N - 1, jnp.int32)
    _, s = jax.lax.fori_loop(0, 12, body2, (lo0i, hi0i))

    c = c_ref[0, 0]
    mask = gt | (eq & (idx <= s))
    out_ref[0] = jnp.where(mask, c, jnp.float32(0.0))


def kernel(param, scalar, noise, b):
    del b
    bsz = noise.shape[0]
    c = ((jnp.float32(1.0) / (jnp.sqrt(jnp.float32(_D)) + jnp.float32(0.01)))
         * jnp.maximum(jnp.float32(0.01), scalar[0])).reshape(1, 1)
    out = pl.pallas_call(
        _mask_kernel,
        grid=(bsz,),
        in_specs=[
            pl.BlockSpec((1, 1), lambda i: (0, 0)),
            pl.BlockSpec((1, _M, _N), lambda i: (0, 0, 0)),
            pl.BlockSpec((1, _M, _N), lambda i: (i, 0, 0)),
        ],
        out_specs=pl.BlockSpec((1, _M, _N), lambda i: (i, 0, 0)),
        out_shape=jax.ShapeDtypeStruct((bsz, _M, _N), jnp.float32),
    )(c, param, noise)
    return out
